# initial kernel scaffold (unmeasured)
import jax
import jax.numpy as jnp
from jax import lax
from jax.experimental import pallas as pl
from jax.experimental.pallas import tpu as pltpu

N_DEV = 4
EPS = 1e-5


def kernel(x, gamma):
    m, n_loc = x.shape
    n_global = N_DEV * n_loc
    p_sub = 8
    p_lane = m // p_sub

    def body(x_ref, g_ref, out_ref, comm_ref, send_sems, recv_sems):
        my = lax.axis_index("i")

        barrier = pltpu.get_barrier_semaphore()
        for off in range(1, N_DEV):
            pl.semaphore_signal(
                barrier, inc=1,
                device_id=((my + off) % N_DEV,),
                device_id_type=pl.DeviceIdType.MESH,
            )
        pl.semaphore_wait(barrier, N_DEV - 1)

        xf = x_ref[:, :].astype(jnp.float32)
        partial = jnp.sum(xf * xf, axis=1)
        comm_ref[my, :, :] = partial.reshape(p_sub, p_lane)

        sends = []
        for k in range(1, N_DEV):
            rdma = pltpu.make_async_remote_copy(
                src_ref=comm_ref.at[my],
                dst_ref=comm_ref.at[my],
                send_sem=send_sems.at[k - 1],
                recv_sem=recv_sems.at[my],
                device_id=((my + k) % N_DEV,),
                device_id_type=pl.DeviceIdType.MESH,
            )
            rdma.start()
            sends.append(rdma)

        for off in range(1, N_DEV):
            src = (my + off) % N_DEV
            recv = pltpu.make_async_remote_copy(
                src_ref=comm_ref.at[src],
                dst_ref=comm_ref.at[src],
                send_sem=send_sems.at[0],
                recv_sem=recv_sems.at[src],
                device_id=(my,),
                device_id_type=pl.DeviceIdType.MESH,
            )
            recv.wait_recv()

        total = comm_ref[0] + comm_ref[1] + comm_ref[2] + comm_ref[3]
        inv = lax.rsqrt(total / n_global + EPS).reshape(m, 1)

        g = g_ref[:].astype(jnp.float32).reshape(1, n_loc)
        out_ref[:, :] = (xf * inv * g).astype(out_ref.dtype)

        for rdma in sends:
            rdma.wait_send()

    return pl.pallas_call(
        body,
        out_shape=jax.ShapeDtypeStruct((m, n_loc), jnp.bfloat16),
        in_specs=[
            pl.BlockSpec(memory_space=pltpu.VMEM),
            pl.BlockSpec(memory_space=pltpu.VMEM),
        ],
        out_specs=pl.BlockSpec(memory_space=pltpu.VMEM),
        scratch_shapes=[
            pltpu.VMEM((N_DEV, p_sub, p_lane), jnp.float32),
            pltpu.SemaphoreType.DMA((N_DEV - 1,)),
            pltpu.SemaphoreType.DMA((N_DEV,)),
        ],
        compiler_params=pltpu.CompilerParams(collective_id=0),
    )(x, gamma)


# baseline (device time: 62234 ns/iter reference)
import jax
import jax.numpy as jnp
from jax import lax
from jax.experimental import pallas as pl
from jax.experimental.pallas import tpu as pltpu

N_DEV = 4
EPS = 1e-5
CH = 256
BLK = 256


def kernel(x, gamma):
    m, n_loc = x.shape
    n_global = N_DEV * n_loc
    n_ch = m // CH
    n_blk = m // BLK

    def body(x_hbm, g_ref, out_hbm, xbf, inbuf, outbuf, comm_ref,
             in_sems, out_sems, send_sems, recv_sems):
        my = lax.axis_index("i")

        barrier = pltpu.get_barrier_semaphore()
        for off in range(1, N_DEV):
            pl.semaphore_signal(
                barrier, inc=1,
                device_id=((my + off) % N_DEV,),
                device_id_type=pl.DeviceIdType.MESH,
            )

        def in_dma(i):
            return pltpu.make_async_copy(
                x_hbm.at[pl.ds(i * CH, CH), :],
                inbuf.at[i % 2],
                in_sems.at[i % 2],
            )

        in_dma(0).start()
        for i in range(n_ch):
            if i + 1 < n_ch:
                in_dma(i + 1).start()
            in_dma(i).wait()
            xc = inbuf[i % 2]
            psum = jnp.sum(xc * xc, axis=1)
            comm_ref[my, :, pl.ds(i * CH, CH)] = psum.reshape(1, CH)
            xbf[pl.ds(i * CH, CH), :] = xc.astype(jnp.bfloat16)

        pl.semaphore_wait(barrier, N_DEV - 1)
        sends = []
        for k in range(1, N_DEV):
            rdma = pltpu.make_async_remote_copy(
                src_ref=comm_ref.at[my],
                dst_ref=comm_ref.at[my],
                send_sem=send_sems.at[k - 1],
                recv_sem=recv_sems.at[my],
                device_id=((my + k) % N_DEV,),
                device_id_type=pl.DeviceIdType.MESH,
            )
            rdma.start()
            sends.append(rdma)
        for off in range(1, N_DEV):
            src = (my + off) % N_DEV
            recv = pltpu.make_async_remote_copy(
                src_ref=comm_ref.at[src],
                dst_ref=comm_ref.at[src],
                send_sem=send_sems.at[0],
                recv_sem=recv_sems.at[src],
                device_id=(my,),
                device_id_type=pl.DeviceIdType.MESH,
            )
            recv.wait_recv()

        total = (comm_ref[0] + comm_ref[1] + comm_ref[2] + comm_ref[3])
        inv_row = lax.rsqrt(total / n_global + EPS)

        g = g_ref[:].astype(jnp.float32).reshape(1, n_loc)
        rowi = lax.broadcasted_iota(jnp.int32, (BLK, BLK), 0)
        coli = lax.broadcasted_iota(jnp.int32, (BLK, BLK), 1)
        diag = (rowi == coli).astype(jnp.float32)

        out_dmas = []
        for b in range(n_blk):
            if b >= 2:
                out_dmas[b - 2].wait()
            v = inv_row[0:1, b * BLK:(b + 1) * BLK]
            inv_col = jnp.sum(diag * v, axis=1, keepdims=True)
            xb = xbf[pl.ds(b * BLK, BLK), :].astype(jnp.float32)
            outbuf[b % 2] = (xb * inv_col * g).astype(jnp.bfloat16)
            dma = pltpu.make_async_copy(
                outbuf.at[b % 2],
                out_hbm.at[pl.ds(b * BLK, BLK), :],
                out_sems.at[b % 2],
            )
            dma.start()
            out_dmas.append(dma)
        out_dmas[-2].wait()
        out_dmas[-1].wait()

        for rdma in sends:
            rdma.wait_send()

    return pl.pallas_call(
        body,
        out_shape=jax.ShapeDtypeStruct((m, n_loc), jnp.bfloat16),
        in_specs=[
            pl.BlockSpec(memory_space=pl.ANY),
            pl.BlockSpec(memory_space=pltpu.VMEM),
        ],
        out_specs=pl.BlockSpec(memory_space=pl.ANY),
        scratch_shapes=[
            pltpu.VMEM((m, n_loc), jnp.bfloat16),
            pltpu.VMEM((2, CH, n_loc), jnp.float32),
            pltpu.VMEM((2, BLK, n_loc), jnp.bfloat16),
            pltpu.VMEM((N_DEV, 1, m), jnp.float32),
            pltpu.SemaphoreType.DMA((2,)),
            pltpu.SemaphoreType.DMA((2,)),
            pltpu.SemaphoreType.DMA((N_DEV - 1,)),
            pltpu.SemaphoreType.DMA((N_DEV,)),
        ],
        compiler_params=pltpu.CompilerParams(collective_id=0),
    )(x, gamma)


# device time: 60276 ns/iter; 1.0325x vs baseline; 1.0325x over previous
import jax
import jax.numpy as jnp
from jax import lax
from jax.experimental import pallas as pl
from jax.experimental.pallas import tpu as pltpu

N_DEV = 4
EPS = 1e-5
CH = 256


def kernel(x, gamma):
    m, n_loc = x.shape
    n_global = N_DEV * n_loc
    n_ch = m // CH

    def body(x_hbm, g_ref, out_hbm, inbuf, outbuf, comm_ref,
             in_sems, out_sems, send_sems, recv_sems):
        my = lax.axis_index("i")

        barrier = pltpu.get_barrier_semaphore()
        for off in range(1, N_DEV):
            pl.semaphore_signal(
                barrier, inc=1,
                device_id=((my + off) % N_DEV,),
                device_id_type=pl.DeviceIdType.MESH,
            )

        NBUF = 4

        def in_dma(j):
            return pltpu.make_async_copy(
                x_hbm.at[pl.ds((j % n_ch) * CH, CH), :],
                inbuf.at[j % NBUF],
                in_sems.at[j % NBUF],
            )

        for j in range(min(NBUF, n_ch)):
            in_dma(j).start()
        for i in range(n_ch):
            in_dma(i).wait()
            xc = inbuf[i % NBUF]
            psum = jnp.sum(xc * xc, axis=1)
            comm_ref[my, :, pl.ds(i * CH, CH)] = psum.reshape(1, CH)
            in_dma(i + NBUF).start()

        pl.semaphore_wait(barrier, N_DEV - 1)
        sends = []
        for k in range(1, N_DEV):
            rdma = pltpu.make_async_remote_copy(
                src_ref=comm_ref.at[my],
                dst_ref=comm_ref.at[my],
                send_sem=send_sems.at[k - 1],
                recv_sem=recv_sems.at[my],
                device_id=((my + k) % N_DEV,),
                device_id_type=pl.DeviceIdType.MESH,
            )
            rdma.start()
            sends.append(rdma)
        for off in range(1, N_DEV):
            src = (my + off) % N_DEV
            recv = pltpu.make_async_remote_copy(
                src_ref=comm_ref.at[src],
                dst_ref=comm_ref.at[src],
                send_sem=send_sems.at[0],
                recv_sem=recv_sems.at[src],
                device_id=(my,),
                device_id_type=pl.DeviceIdType.MESH,
            )
            recv.wait_recv()

        total = (comm_ref[0] + comm_ref[1] + comm_ref[2] + comm_ref[3])
        inv_row = lax.rsqrt(total / n_global + EPS)

        g = g_ref[:].astype(jnp.float32).reshape(1, n_loc)
        rowi = lax.broadcasted_iota(jnp.int32, (CH, CH), 0)
        coli = lax.broadcasted_iota(jnp.int32, (CH, CH), 1)
        diag = (rowi == coli).astype(jnp.float32)

        out_dmas = []
        for b in range(n_ch):
            if b >= 2:
                out_dmas[b - 2].wait()
            in_dma(n_ch + b).wait()
            v = inv_row[0:1, b * CH:(b + 1) * CH]
            inv_col = jnp.sum(diag * v, axis=1, keepdims=True)
            xc = inbuf[(n_ch + b) % NBUF]
            outbuf[b % 2] = (xc * inv_col * g).astype(jnp.bfloat16)
            if n_ch + b + NBUF < 2 * n_ch:
                in_dma(n_ch + b + NBUF).start()
            dma = pltpu.make_async_copy(
                outbuf.at[b % 2],
                out_hbm.at[pl.ds(b * CH, CH), :],
                out_sems.at[b % 2],
            )
            dma.start()
            out_dmas.append(dma)
        out_dmas[-2].wait()
        out_dmas[-1].wait()

        for rdma in sends:
            rdma.wait_send()

    return pl.pallas_call(
        body,
        out_shape=jax.ShapeDtypeStruct((m, n_loc), jnp.bfloat16),
        in_specs=[
            pl.BlockSpec(memory_space=pl.ANY),
            pl.BlockSpec(memory_space=pltpu.VMEM),
        ],
        out_specs=pl.BlockSpec(memory_space=pl.ANY),
        scratch_shapes=[
            pltpu.VMEM((4, CH, n_loc), jnp.float32),
            pltpu.VMEM((2, CH, n_loc), jnp.bfloat16),
            pltpu.VMEM((N_DEV, 1, m), jnp.float32),
            pltpu.SemaphoreType.DMA((4,)),
            pltpu.SemaphoreType.DMA((2,)),
            pltpu.SemaphoreType.DMA((N_DEV - 1,)),
            pltpu.SemaphoreType.DMA((N_DEV,)),
        ],
        compiler_params=pltpu.CompilerParams(collective_id=0),
    )(x, gamma)
